# baseline (speedup 1.0000x reference)
import functools

import jax
import jax.numpy as jnp
from jax import lax
from jax.experimental import pallas as pl
from jax.experimental.pallas import tpu as pltpu
from jax.experimental.pallas import tpu_sc as plsc

N = 10000
D = 128
DG = 16
E = 320000
L = 320000
CA = 80
NA = 125
EPW = 10000

_info = plsc.get_sparse_core_info()
NC, NS, LANES = _info.num_cores, _info.num_subcores, _info.num_lanes
NW = NC * NS
RPT = N // NS


def _sc_agg(x, src3, dst3, zeros, *, with_deg):
    def body(*refs):
        if with_deg:
            (x_hbm, src3_hbm, dst3_hbm, zero_hbm, feat_hbm, deg_hbm,
             dst_st, srcb0, srcb1, rows0, rows1, ones_v, acc_sh,
             dacc_sh, isem, gsem, ssem) = refs
        else:
            (x_hbm, src3_hbm, dst3_hbm, zero_hbm, feat_hbm,
             dst_st, srcb0, srcb1, rows0, rows1, acc_sh,
             isem, gsem, ssem) = refs
        c = lax.axis_index("c")
        s = lax.axis_index("s")
        wid = s * NC + c
        rows = (rows0, rows1)
        srcb = (srcb0, srcb1)

        pltpu.sync_copy(zero_hbm.at[pl.ds(s * RPT, RPT)],
                        acc_sh.at[pl.ds(s * RPT, RPT)])
        if with_deg:
            pltpu.sync_copy(zero_hbm.at[pl.ds(s * RPT, RPT), pl.ds(0, DG)],
                            dacc_sh.at[pl.ds(s * RPT, RPT)])

            def fill_ones(r, carry):
                ones_v[r, :] = jnp.full((LANES,), 1.0, jnp.float32)
                return carry

            lax.fori_loop(0, CA, fill_ones, 0)

        pltpu.sync_copy(dst3_hbm.at[wid], dst_st)

        def i_issue(i, b):
            pltpu.async_copy(src3_hbm.at[wid, i], srcb[b], isem)

        def i_wait(i, b):
            pltpu.make_async_copy(src3_hbm.at[wid, i], srcb[b], isem).wait()

        def g_issue(i, b):
            pltpu.async_copy(x_hbm.at[srcb[b]], rows[b], gsem)

        def g_wait(i, b):
            pltpu.make_async_copy(x_hbm.at[srcb[b]], rows[b], gsem).wait()

        def s_issue(i, b):
            pltpu.async_copy(rows[b], acc_sh.at[dst_st.at[i]], ssem, add=True)
            if with_deg:
                pltpu.async_copy(ones_v, dacc_sh.at[dst_st.at[i]], ssem,
                                 add=True)

        def s_wait(i, b):
            pltpu.make_async_copy(rows[b], acc_sh.at[dst_st.at[i]],
                                  ssem).wait()
            if with_deg:
                pltpu.make_async_copy(ones_v, dacc_sh.at[dst_st.at[i]],
                                      ssem).wait()

        i_issue(0, 0)
        i_wait(0, 0)
        g_issue(0, 0)
        i_issue(1, 1)
        plsc.subcore_barrier()

        def half(i, b):
            g_wait(i, b)

            @pl.when(i >= 1)
            def _():
                s_wait(i - 1, 1 - b)

            @pl.when(i <= NA - 2)
            def _():
                i_wait(i + 1, 1 - b)
                g_issue(i + 1, 1 - b)

            @pl.when(i <= NA - 3)
            def _():
                i_issue(i + 2, b)

            s_issue(i, b)

        half(0, 0)

        def pair(k, carry):
            half(2 * k + 1, 1)
            half(2 * k + 2, 0)
            return carry

        lax.fori_loop(0, (NA - 1) // 2, pair, 0)
        s_wait(NA - 1, 0)
        plsc.subcore_barrier()

        pltpu.sync_copy(acc_sh.at[pl.ds(s * RPT, RPT)],
                        feat_hbm.at[c, pl.ds(s * RPT, RPT)])
        if with_deg:
            pltpu.sync_copy(dacc_sh.at[pl.ds(s * RPT, RPT)],
                            deg_hbm.at[c, pl.ds(s * RPT, RPT)])

    out_type = [jax.ShapeDtypeStruct((NC, N, D), jnp.float32)]
    scratch = [
        pltpu.VMEM((NA, CA), jnp.int32),
        pltpu.VMEM((CA,), jnp.int32),
        pltpu.VMEM((CA,), jnp.int32),
        pltpu.VMEM((CA, D), jnp.float32),
        pltpu.VMEM((CA, D), jnp.float32),
    ]
    if with_deg:
        out_type.append(jax.ShapeDtypeStruct((NC, N, DG), jnp.float32))
        scratch.append(pltpu.VMEM((CA, DG), jnp.float32))
    scratch.append(pltpu.VMEM_SHARED((N, D), jnp.float32))
    if with_deg:
        scratch.append(pltpu.VMEM_SHARED((N, DG), jnp.float32))
    scratch += [pltpu.SemaphoreType.DMA] * 3

    mesh = plsc.VectorSubcoreMesh(core_axis_name="c", subcore_axis_name="s")
    f = pl.kernel(
        body,
        mesh=mesh,
        compiler_params=pltpu.CompilerParams(use_tc_tiling_on_sc=False),
        out_type=tuple(out_type),
        scratch_types=scratch,
    )
    return f(x, src3, dst3, zeros)


def _sc_linkpred_body(h_hbm, a2_hbm, b2_hbm, out3_hbm,
                      a_st, b_st, ra0, ra1, rb0, rb1, tmp_v, res0, res1,
                      sema, semb, wsem):
    c = lax.axis_index("c")
    s = lax.axis_index("s")
    wid = s * NC + c
    ra = (ra0, ra1)
    rb = (rb0, rb1)
    res = (res0, res1)
    lane = jnp.arange(LANES, dtype=jnp.int32)

    pltpu.sync_copy(a2_hbm.at[wid], a_st)
    pltpu.sync_copy(b2_hbm.at[wid], b_st)

    def g_issue(i, b):
        pltpu.async_copy(h_hbm.at[a_st.at[pl.ds(i * CA, CA)]], ra[b], sema)
        pltpu.async_copy(h_hbm.at[b_st.at[pl.ds(i * CA, CA)]], rb[b], semb)

    def g_wait(i, b):
        pltpu.make_async_copy(h_hbm.at[a_st.at[pl.ds(i * CA, CA)]],
                              ra[b], sema).wait()
        pltpu.make_async_copy(h_hbm.at[b_st.at[pl.ds(i * CA, CA)]],
                              rb[b], semb).wait()

    def w_issue(i, b):
        pltpu.async_copy(res[b], out3_hbm.at[wid, i], wsem)

    def w_wait(i, b):
        pltpu.make_async_copy(res[b], out3_hbm.at[wid, i], wsem).wait()

    g_issue(0, 0)

    def half(i, b):
        g_wait(i, b)

        @pl.when(i <= NA - 2)
        def _():
            g_issue(i + 1, 1 - b)

        def group(g, c2):
            for e in range(LANES):
                row = g * LANES + e
                acc = jnp.zeros((LANES,), jnp.float32)
                for j in range(D // (2 * LANES)):
                    prod = (ra[b][row, pl.ds(j * 2 * LANES, 2 * LANES)]
                            * rb[b][row, pl.ds(j * 2 * LANES, 2 * LANES)])
                    lo, hi = plsc.unpack(prod,
                                         format=plsc.PackFormat.INTERLEAVED)
                    acc = acc + lo + hi
                tmp_v[e, :] = acc
            tot = jnp.zeros((LANES,), jnp.float32)
            for l in range(LANES):
                col = jnp.full((LANES,), l, jnp.int32)
                tot = tot + plsc.load_gather(tmp_v, [lane, col])
            res[b][pl.ds(g * LANES, LANES)] = tot
            return c2

        lax.fori_loop(0, CA // LANES, group, 0)

        @pl.when(i >= 1)
        def _():
            w_wait(i - 1, 1 - b)

        w_issue(i, b)

    half(0, 0)

    def pair(k, carry):
        half(2 * k + 1, 1)
        half(2 * k + 2, 0)
        return carry

    lax.fori_loop(0, (NA - 1) // 2, pair, 0)
    w_wait(NA - 1, 0)


def _sc_linkpred(h, a2, b2):
    mesh = plsc.VectorSubcoreMesh(core_axis_name="c", subcore_axis_name="s")
    f = pl.kernel(
        _sc_linkpred_body,
        mesh=mesh,
        compiler_params=pltpu.CompilerParams(use_tc_tiling_on_sc=False,
                                             needs_layout_passes=False),
        out_type=jax.ShapeDtypeStruct((NW, NA, CA), jnp.float32),
        scratch_types=[
            pltpu.VMEM((EPW,), jnp.int32),
            pltpu.VMEM((EPW,), jnp.int32),
            pltpu.VMEM((CA, D), jnp.bfloat16),
            pltpu.VMEM((CA, D), jnp.bfloat16),
            pltpu.VMEM((CA, D), jnp.bfloat16),
            pltpu.VMEM((CA, D), jnp.bfloat16),
            pltpu.VMEM((LANES, LANES), jnp.float32),
            pltpu.VMEM((CA,), jnp.float32),
            pltpu.VMEM((CA,), jnp.float32),
            pltpu.SemaphoreType.DMA,
            pltpu.SemaphoreType.DMA,
            pltpu.SemaphoreType.DMA,
        ],
    )
    return f(h, a2, b2)


def _tc_dense_body(with_relu, f_ref, dg_ref, x_ref, wl_ref, bl_ref, wr_ref,
                   g_ref, be_ref, out_ref):
    p = f_ref[0] + f_ref[1]
    deg = jnp.maximum(dg_ref[0, :, 0:1] + dg_ref[1, :, 0:1], 1.0)
    mean = p / deg
    z = lax.dot_general(mean, wl_ref[...], (((1,), (1,)), ((), ())),
                        preferred_element_type=jnp.float32)
    z = z + bl_ref[...][None, :]
    z = z + lax.dot_general(x_ref[...], wr_ref[...], (((1,), (1,)), ((), ())),
                            preferred_element_type=jnp.float32)
    m = jnp.mean(z, axis=0, keepdims=True)
    v = jnp.mean((z - m) * (z - m), axis=0, keepdims=True)
    h = (z - m) * lax.rsqrt(v + 1e-5) * g_ref[...][None, :] + be_ref[...][None, :]
    if with_relu:
        h = jnp.where(h >= 0, h, 0.01 * h)
    out_ref[...] = h.astype(out_ref.dtype)


def _tc_dense(feat, deg, x, wl, bl, wr, g, be, *, with_relu, out_dtype):
    return pl.pallas_call(
        functools.partial(_tc_dense_body, with_relu),
        out_shape=jax.ShapeDtypeStruct((N, D), out_dtype),
    )(feat, deg, x, wl, bl, wr, g, be)


def kernel(node_feature, edge_index, edge_label_index,
           W1l, b1l, W1r, g1, be1, W2l, b2l, W2r, g2, be2):
    src3 = edge_index[0].astype(jnp.int32).reshape(NW, NA, CA)
    dst3 = edge_index[1].astype(jnp.int32).reshape(NW, NA, CA)
    a2 = edge_label_index[0].astype(jnp.int32).reshape(NW, EPW)
    b2 = edge_label_index[1].astype(jnp.int32).reshape(NW, EPW)
    zeros = jnp.zeros((N, D), jnp.float32)

    feat1, deg = _sc_agg(node_feature, src3, dst3, zeros, with_deg=True)
    h1 = _tc_dense(feat1, deg, node_feature, W1l, b1l, W1r, g1, be1,
                   with_relu=True, out_dtype=jnp.float32)
    (feat2,) = _sc_agg(h1, src3, dst3, zeros, with_deg=False)
    h2 = _tc_dense(feat2, deg, h1, W2l, b2l, W2r, g2, be2, with_relu=False,
                   out_dtype=jnp.bfloat16)
    pred = _sc_linkpred(h2, a2, b2)
    return pred.reshape(L)


# speedup vs baseline: 1.1157x; 1.1157x over previous
import functools

import jax
import jax.numpy as jnp
from jax import lax
from jax.experimental import pallas as pl
from jax.experimental.pallas import tpu as pltpu
from jax.experimental.pallas import tpu_sc as plsc

N = 10000
D = 128
DG = 16
E = 320000
L = 320000
CA = 80
NA = 125
EPW = 10000

_info = plsc.get_sparse_core_info()
NC, NS, LANES = _info.num_cores, _info.num_subcores, _info.num_lanes
NW = NC * NS
RPT = N // NS


def _sc_agg(x, src3, dst3, zeros, *, with_deg):
    def body(*refs):
        if with_deg:
            (x_hbm, src3_hbm, dst3_hbm, zero_hbm, feat_hbm, deg_hbm,
             dst_st, srcb0, srcb1, rows0, rows1, ones_v, acc_sh,
             dacc_sh, isem, gsem, ssem) = refs
        else:
            (x_hbm, src3_hbm, dst3_hbm, zero_hbm, feat_hbm,
             dst_st, srcb0, srcb1, rows0, rows1, acc_sh,
             isem, gsem, ssem) = refs
        c = lax.axis_index("c")
        s = lax.axis_index("s")
        wid = s * NC + c
        rows = (rows0, rows1)
        srcb = (srcb0, srcb1)

        pltpu.sync_copy(zero_hbm.at[pl.ds(s * RPT, RPT)],
                        acc_sh.at[pl.ds(s * RPT, RPT)])
        if with_deg:
            pltpu.sync_copy(zero_hbm.at[pl.ds(s * RPT, RPT), pl.ds(0, DG)],
                            dacc_sh.at[pl.ds(s * RPT, RPT)])

            def fill_ones(r, carry):
                ones_v[r, :] = jnp.full((LANES,), 1.0, jnp.float32)
                return carry

            lax.fori_loop(0, CA, fill_ones, 0)

        pltpu.sync_copy(dst3_hbm.at[wid], dst_st)

        def i_issue(i, b):
            pltpu.async_copy(src3_hbm.at[wid, i], srcb[b], isem)

        def i_wait(i, b):
            pltpu.make_async_copy(src3_hbm.at[wid, i], srcb[b], isem).wait()

        def g_issue(i, b):
            pltpu.async_copy(x_hbm.at[srcb[b]], rows[b], gsem)

        def g_wait(i, b):
            pltpu.make_async_copy(x_hbm.at[srcb[b]], rows[b], gsem).wait()

        def s_issue(i, b):
            pltpu.async_copy(rows[b], acc_sh.at[dst_st.at[i]], ssem, add=True)
            if with_deg:
                pltpu.async_copy(ones_v, dacc_sh.at[dst_st.at[i]], ssem,
                                 add=True)

        def s_wait(i, b):
            pltpu.make_async_copy(rows[b], acc_sh.at[dst_st.at[i]],
                                  ssem).wait()
            if with_deg:
                pltpu.make_async_copy(ones_v, dacc_sh.at[dst_st.at[i]],
                                      ssem).wait()

        i_issue(0, 0)
        i_wait(0, 0)
        g_issue(0, 0)
        i_issue(1, 1)
        plsc.subcore_barrier()

        def half(i, b):
            g_wait(i, b)

            @pl.when(i >= 1)
            def _():
                s_wait(i - 1, 1 - b)

            @pl.when(i <= NA - 2)
            def _():
                i_wait(i + 1, 1 - b)
                g_issue(i + 1, 1 - b)

            @pl.when(i <= NA - 3)
            def _():
                i_issue(i + 2, b)

            s_issue(i, b)

        half(0, 0)

        def pair(k, carry):
            half(2 * k + 1, 1)
            half(2 * k + 2, 0)
            return carry

        lax.fori_loop(0, (NA - 1) // 2, pair, 0)
        s_wait(NA - 1, 0)
        plsc.subcore_barrier()

        pltpu.sync_copy(acc_sh.at[pl.ds(s * RPT, RPT)],
                        feat_hbm.at[c, pl.ds(s * RPT, RPT)])
        if with_deg:
            pltpu.sync_copy(dacc_sh.at[pl.ds(s * RPT, RPT)],
                            deg_hbm.at[c, pl.ds(s * RPT, RPT)])

    out_type = [jax.ShapeDtypeStruct((NC, N, D), jnp.float32)]
    scratch = [
        pltpu.VMEM((NA, CA), jnp.int32),
        pltpu.VMEM((CA,), jnp.int32),
        pltpu.VMEM((CA,), jnp.int32),
        pltpu.VMEM((CA, D), jnp.float32),
        pltpu.VMEM((CA, D), jnp.float32),
    ]
    if with_deg:
        out_type.append(jax.ShapeDtypeStruct((NC, N, DG), jnp.float32))
        scratch.append(pltpu.VMEM((CA, DG), jnp.float32))
    scratch.append(pltpu.VMEM_SHARED((N, D), jnp.float32))
    if with_deg:
        scratch.append(pltpu.VMEM_SHARED((N, DG), jnp.float32))
    scratch += [pltpu.SemaphoreType.DMA] * 3

    mesh = plsc.VectorSubcoreMesh(core_axis_name="c", subcore_axis_name="s")
    f = pl.kernel(
        body,
        mesh=mesh,
        compiler_params=pltpu.CompilerParams(use_tc_tiling_on_sc=False),
        out_type=tuple(out_type),
        scratch_types=scratch,
    )
    return f(x, src3, dst3, zeros)


def _sc_linkpred_body(h_hbm, a2_hbm, b2_hbm, out3_hbm,
                      a_st, b_st, ra0, ra1, rb0, rb1, tmp_v, res0, res1,
                      sema, semb, wsem):
    c = lax.axis_index("c")
    s = lax.axis_index("s")
    wid = s * NC + c
    ra = (ra0, ra1)
    rb = (rb0, rb1)
    res = (res0, res1)
    lane = jnp.arange(LANES, dtype=jnp.int32)

    pltpu.sync_copy(a2_hbm.at[wid], a_st)
    pltpu.sync_copy(b2_hbm.at[wid], b_st)

    def g_issue(i, b):
        pltpu.async_copy(h_hbm.at[a_st.at[pl.ds(i * CA, CA)]], ra[b], sema)
        pltpu.async_copy(h_hbm.at[b_st.at[pl.ds(i * CA, CA)]], rb[b], semb)

    def g_wait(i, b):
        pltpu.make_async_copy(h_hbm.at[a_st.at[pl.ds(i * CA, CA)]],
                              ra[b], sema).wait()
        pltpu.make_async_copy(h_hbm.at[b_st.at[pl.ds(i * CA, CA)]],
                              rb[b], semb).wait()

    def w_issue(i, b):
        pltpu.async_copy(res[b], out3_hbm.at[wid, i], wsem)

    def w_wait(i, b):
        pltpu.make_async_copy(res[b], out3_hbm.at[wid, i], wsem).wait()

    g_issue(0, 0)

    def half(i, b):
        g_wait(i, b)

        @pl.when(i <= NA - 2)
        def _():
            g_issue(i + 1, 1 - b)

        @plsc.parallel_loop(0, CA, unroll=4)
        def _edge(e):
            acc = jnp.zeros((LANES,), jnp.float32)
            for j in range(D // (2 * LANES)):
                prod = (ra[b][e, pl.ds(j * 2 * LANES, 2 * LANES)]
                        * rb[b][e, pl.ds(j * 2 * LANES, 2 * LANES)])
                lo, hi = plsc.unpack(prod,
                                     format=plsc.PackFormat.INTERLEAVED)
                acc = acc + lo + hi
            tmp_v[e, :] = acc

        @plsc.parallel_loop(0, CA // LANES, unroll=5)
        def _group(g):
            tot = jnp.zeros((LANES,), jnp.float32)
            rowi = lane + g * LANES
            for l in range(LANES):
                col = jnp.full((LANES,), l, jnp.int32)
                tot = tot + plsc.load_gather(tmp_v, [rowi, col])
            res[b][pl.ds(g * LANES, LANES)] = tot

        @pl.when(i >= 1)
        def _():
            w_wait(i - 1, 1 - b)

        w_issue(i, b)

    half(0, 0)

    def pair(k, carry):
        half(2 * k + 1, 1)
        half(2 * k + 2, 0)
        return carry

    lax.fori_loop(0, (NA - 1) // 2, pair, 0)
    w_wait(NA - 1, 0)


def _sc_linkpred(h, a2, b2):
    mesh = plsc.VectorSubcoreMesh(core_axis_name="c", subcore_axis_name="s")
    f = pl.kernel(
        _sc_linkpred_body,
        mesh=mesh,
        compiler_params=pltpu.CompilerParams(use_tc_tiling_on_sc=False,
                                             needs_layout_passes=False),
        out_type=jax.ShapeDtypeStruct((NW, NA, CA), jnp.float32),
        scratch_types=[
            pltpu.VMEM((EPW,), jnp.int32),
            pltpu.VMEM((EPW,), jnp.int32),
            pltpu.VMEM((CA, D), jnp.bfloat16),
            pltpu.VMEM((CA, D), jnp.bfloat16),
            pltpu.VMEM((CA, D), jnp.bfloat16),
            pltpu.VMEM((CA, D), jnp.bfloat16),
            pltpu.VMEM((CA, LANES), jnp.float32),
            pltpu.VMEM((CA,), jnp.float32),
            pltpu.VMEM((CA,), jnp.float32),
            pltpu.SemaphoreType.DMA,
            pltpu.SemaphoreType.DMA,
            pltpu.SemaphoreType.DMA,
        ],
    )
    return f(h, a2, b2)


def _tc_dense_body(with_relu, f_ref, dg_ref, x_ref, wl_ref, bl_ref, wr_ref,
                   g_ref, be_ref, out_ref):
    p = f_ref[0] + f_ref[1]
    deg = jnp.maximum(dg_ref[0, :, 0:1] + dg_ref[1, :, 0:1], 1.0)
    mean = p / deg
    z = lax.dot_general(mean, wl_ref[...], (((1,), (1,)), ((), ())),
                        preferred_element_type=jnp.float32)
    z = z + bl_ref[...][None, :]
    z = z + lax.dot_general(x_ref[...], wr_ref[...], (((1,), (1,)), ((), ())),
                            preferred_element_type=jnp.float32)
    m = jnp.mean(z, axis=0, keepdims=True)
    v = jnp.mean((z - m) * (z - m), axis=0, keepdims=True)
    h = (z - m) * lax.rsqrt(v + 1e-5) * g_ref[...][None, :] + be_ref[...][None, :]
    if with_relu:
        h = jnp.where(h >= 0, h, 0.01 * h)
    out_ref[...] = h.astype(out_ref.dtype)


def _tc_dense(feat, deg, x, wl, bl, wr, g, be, *, with_relu, out_dtype):
    return pl.pallas_call(
        functools.partial(_tc_dense_body, with_relu),
        out_shape=jax.ShapeDtypeStruct((N, D), out_dtype),
    )(feat, deg, x, wl, bl, wr, g, be)


def kernel(node_feature, edge_index, edge_label_index,
           W1l, b1l, W1r, g1, be1, W2l, b2l, W2r, g2, be2):
    src3 = edge_index[0].astype(jnp.int32).reshape(NW, NA, CA)
    dst3 = edge_index[1].astype(jnp.int32).reshape(NW, NA, CA)
    a2 = edge_label_index[0].astype(jnp.int32).reshape(NW, EPW)
    b2 = edge_label_index[1].astype(jnp.int32).reshape(NW, EPW)
    zeros = jnp.zeros((N, D), jnp.float32)

    feat1, deg = _sc_agg(node_feature, src3, dst3, zeros, with_deg=True)
    h1 = _tc_dense(feat1, deg, node_feature, W1l, b1l, W1r, g1, be1,
                   with_relu=True, out_dtype=jnp.float32)
    (feat2,) = _sc_agg(h1, src3, dst3, zeros, with_deg=False)
    h2 = _tc_dense(feat2, deg, h1, W2l, b2l, W2r, g2, be2, with_relu=False,
                   out_dtype=jnp.bfloat16)
    pred = _sc_linkpred(h2, a2, b2)
    return pred.reshape(L)
